# baseline (speedup 1.0000x reference)
import jax
import jax.numpy as jnp
from jax import lax
from jax.experimental import pallas as pl
from jax.experimental.pallas import tpu as pltpu
from jax.experimental.pallas import tpu_sc as plsc

_BBOX_NUM = 300
_NMS_THRESHOLD = 0.5
_N = 20000
_N_PAD = 20480
_B = 8
_PARTS = 4
_SHARD = _N_PAD // _PARTS
_CHUNKS = _SHARD // 16
_NEG = float("-inf")
_BIGPOS = 2 ** 30


def _f2i(x):
    return lax.bitcast_convert_type(x, jnp.int32)


def _i2f(x):
    return lax.bitcast_convert_type(x, jnp.float32)


def _rep(ref, idx):
    if not (hasattr(idx, "shape") and idx.shape == (16,)):
        idx = jnp.full((16,), idx, jnp.int32)
    return plsc.load_gather(ref, [idx])


_L15 = None


def _bcast_last(v):
    idx = jnp.full((16,), 15, jnp.int32)
    return v.at[idx].get(mode="promise_in_bounds")


def _sc_body(score_hbm, y1_hbm, x1_hbm, y2_hbm, x2_hbm, out_hbm,
             score_v, y1_v, x1_v, y2_v, x2_v, area_v,
             stage_v, cand_v, out_v, shared):
    c = lax.axis_index("c")
    s = lax.axis_index("s")
    row = c * (16 // _PARTS) + s // _PARTS
    part = s % _PARTS
    base = part * _SHARD

    pltpu.sync_copy(score_hbm.at[row, pl.ds(base, _SHARD)], score_v)
    pltpu.sync_copy(y1_hbm.at[row, pl.ds(base, _SHARD)], y1_v)
    pltpu.sync_copy(x1_hbm.at[row, pl.ds(base, _SHARD)], x1_v)
    pltpu.sync_copy(y2_hbm.at[row, pl.ds(base, _SHARD)], y2_v)
    pltpu.sync_copy(x2_hbm.at[row, pl.ds(base, _SHARD)], x2_v)

    lane = lax.iota(jnp.int32, 16)

    def area_body(j, _):
        off = j * 16
        h = jnp.maximum(y2_v[pl.ds(off, 16)] - y1_v[pl.ds(off, 16)], 0.0)
        w = jnp.maximum(x2_v[pl.ds(off, 16)] - x1_v[pl.ds(off, 16)], 0.0)
        area_v[pl.ds(off, 16)] = h * w
        return 0

    lax.fori_loop(0, _CHUNKS, area_body, 0)

    def iter_body(k, carry):
        wy1, wx1, wy2, wx2, wgpos = carry
        warea = jnp.maximum(wy2 - wy1, 0.0) * jnp.maximum(wx2 - wx1, 0.0)

        def chunk_body(j, mc):
            maxvec, posvec = mc
            off = j * 16
            sc = score_v[pl.ds(off, 16)]
            cy1 = y1_v[pl.ds(off, 16)]
            cx1 = x1_v[pl.ds(off, 16)]
            cy2 = y2_v[pl.ds(off, 16)]
            cx2 = x2_v[pl.ds(off, 16)]
            ar = area_v[pl.ds(off, 16)]
            yy1 = jnp.maximum(wy1, cy1)
            xx1 = jnp.maximum(wx1, cx1)
            yy2 = jnp.minimum(wy2, cy2)
            xx2 = jnp.minimum(wx2, cx2)
            inter = jnp.maximum(yy2 - yy1, 0.0) * jnp.maximum(xx2 - xx1, 0.0)
            union = warea + ar - inter
            iou = jnp.where(union > 0.0, inter / union, 0.0)
            pos = lane + (base + off)
            sup = (iou > _NMS_THRESHOLD) | (pos == wgpos)
            nsc = jnp.where(sup, _NEG, sc)
            score_v[pl.ds(off, 16)] = nsc
            upd = nsc > maxvec
            return (jnp.where(upd, nsc, maxvec), jnp.where(upd, pos, posvec))

        maxvec0 = jnp.full((16,), _NEG, jnp.float32)
        posvec0 = jnp.full((16,), base, jnp.int32)
        maxvec, posvec = lax.fori_loop(0, _CHUNKS, chunk_body,
                                       (maxvec0, posvec0))
        lmv = _bcast_last(plsc.cummax(maxvec))
        posm = jnp.where(maxvec == lmv, posvec, _BIGPOS)
        lpv = -_bcast_last(plsc.cummax(-posm))
        liv = lpv - base
        stage = jnp.where(lane == 0, _f2i(lmv), lpv)
        stage = jnp.where(lane == 2, _f2i(_rep(y1_v, liv)), stage)
        stage = jnp.where(lane == 3, _f2i(_rep(x1_v, liv)), stage)
        stage = jnp.where(lane == 4, _f2i(_rep(y2_v, liv)), stage)
        stage = jnp.where(lane == 5, _f2i(_rep(x2_v, liv)), stage)
        stage = jnp.where(lane == 1, lpv, stage)
        stage_v[...] = stage

        pltpu.sync_copy(stage_v, shared.at[0, pl.ds(s * 16, 16)])
        plsc.subcore_barrier()
        g0 = (s // _PARTS) * _PARTS
        pltpu.sync_copy(shared.at[0, pl.ds(g0 * 16, _PARTS * 16)], cand_v)
        plsc.subcore_barrier()

        idx0 = jnp.full((16,), 0, jnp.int32)
        idx1 = jnp.full((16,), 1, jnp.int32)

        def _fld(vec, idx):
            return vec.at[idx].get(mode="promise_in_bounds")

        best = cand_v[pl.ds(0, 16)]
        bm = _i2f(_fld(best, idx0))
        bp = _fld(best, idx1)
        for j in range(1, _PARTS):
            cj = cand_v[pl.ds(j * 16, 16)]
            cm = _i2f(_fld(cj, idx0))
            cp = _fld(cj, idx1)
            better = (cm > bm) | ((cm == bm) & (cp < bp))
            bm = jnp.where(better, cm, bm)
            bp = jnp.where(better, cp, bp)
            best = jnp.where(better, cj, best)
        def _field(i):
            idx = jnp.full((16,), i, jnp.int32)
            return best.at[idx].get(mode="promise_in_bounds")

        valid = _i2f(_field(0)) > _NEG
        wgposn = _field(1)
        wy1n = _i2f(_field(2))
        wx1n = _i2f(_field(3))
        wy2n = _i2f(_field(4))
        wx2n = _i2f(_field(5))

        @pl.when(part == 0)
        def _():
            cidx = jnp.minimum(lane + 2, 15)
            coords = _i2f(best.at[cidx].get(mode="promise_in_bounds"))
            vals = jnp.where(valid, coords, -1.0)
            plsc.store_scatter(out_v, [4 * k + lane], vals, mask=lane < 4)

        return (wy1n, wx1n, wy2n, wx2n, wgposn)

    z = jnp.zeros((16,), jnp.float32)
    init = (z, z, z, z, jnp.full((16,), -1, jnp.int32))
    lax.fori_loop(0, _BBOX_NUM, iter_body, init)

    @pl.when(part == 0)
    def _():
        pltpu.sync_copy(out_v, out_hbm.at[row])


@jax.jit
def _nms_sc(scores, y1, x1, y2, x2):
    mesh = plsc.VectorSubcoreMesh(core_axis_name="c", subcore_axis_name="s")
    return pl.kernel(
        _sc_body,
        out_type=jax.ShapeDtypeStruct((_B, _BBOX_NUM * 4), jnp.float32),
        mesh=mesh,
        compiler_params=pltpu.CompilerParams(needs_layout_passes=False),
        scratch_types=[
            pltpu.VMEM((_SHARD,), jnp.float32),
            pltpu.VMEM((_SHARD,), jnp.float32),
            pltpu.VMEM((_SHARD,), jnp.float32),
            pltpu.VMEM((_SHARD,), jnp.float32),
            pltpu.VMEM((_SHARD,), jnp.float32),
            pltpu.VMEM((_SHARD,), jnp.float32),
            pltpu.VMEM((16,), jnp.int32),
            pltpu.VMEM((_PARTS * 16,), jnp.int32),
            pltpu.VMEM((_BBOX_NUM * 4,), jnp.float32),
            pltpu.VMEM_SHARED((2, 256), jnp.int32),
        ],
    )(scores, y1, x1, y2, x2)


def kernel(classifications, bboxes):
    scores = classifications[:, :, 1]
    scores = jnp.pad(scores, ((0, 0), (0, _N_PAD - _N)), constant_values=_NEG)
    coords = jnp.pad(bboxes, ((0, 0), (0, _N_PAD - _N), (0, 0)))
    y1 = coords[:, :, 0]
    x1 = coords[:, :, 1]
    y2 = coords[:, :, 2]
    x2 = coords[:, :, 3]
    out = _nms_sc(scores, y1, x1, y2, x2)
    return out.reshape(_B, _BBOX_NUM, 4)


# speedup vs baseline: 1.0280x; 1.0280x over previous
import jax
import jax.numpy as jnp
from jax import lax
from jax.experimental import pallas as pl
from jax.experimental.pallas import tpu as pltpu
from jax.experimental.pallas import tpu_sc as plsc

_BBOX_NUM = 300
_NMS_THRESHOLD = 0.5
_N = 20000
_N_PAD = 20480
_B = 8
_PARTS = 4
_SHARD = _N_PAD // _PARTS
_CHUNKS = _SHARD // 16
_NEG = float("-inf")
_BIGPOS = 2 ** 30


def _f2i(x):
    return lax.bitcast_convert_type(x, jnp.int32)


def _i2f(x):
    return lax.bitcast_convert_type(x, jnp.float32)


def _rep(ref, idx):
    if not (hasattr(idx, "shape") and idx.shape == (16,)):
        idx = jnp.full((16,), idx, jnp.int32)
    return plsc.load_gather(ref, [idx])


_L15 = None


def _bcast_last(v):
    idx = jnp.full((16,), 15, jnp.int32)
    return v.at[idx].get(mode="promise_in_bounds")


def _sc_body(score_hbm, y1_hbm, x1_hbm, y2_hbm, x2_hbm, out_hbm,
             score_v, y1_v, x1_v, y2_v, x2_v, area_v,
             stage_v, cand_v, out_v, shared):
    c = lax.axis_index("c")
    s = lax.axis_index("s")
    row = c * (16 // _PARTS) + s // _PARTS
    part = s % _PARTS
    base = part * _SHARD

    pltpu.sync_copy(score_hbm.at[row, pl.ds(base, _SHARD)], score_v)
    pltpu.sync_copy(y1_hbm.at[row, pl.ds(base, _SHARD)], y1_v)
    pltpu.sync_copy(x1_hbm.at[row, pl.ds(base, _SHARD)], x1_v)
    pltpu.sync_copy(y2_hbm.at[row, pl.ds(base, _SHARD)], y2_v)
    pltpu.sync_copy(x2_hbm.at[row, pl.ds(base, _SHARD)], x2_v)

    lane = lax.iota(jnp.int32, 16)

    def area_body(j, _):
        off = j * 16
        h = jnp.maximum(y2_v[pl.ds(off, 16)] - y1_v[pl.ds(off, 16)], 0.0)
        w = jnp.maximum(x2_v[pl.ds(off, 16)] - x1_v[pl.ds(off, 16)], 0.0)
        area_v[pl.ds(off, 16)] = h * w
        return 0

    lax.fori_loop(0, _CHUNKS, area_body, 0)

    def iter_body(k, carry):
        wy1, wx1, wy2, wx2, wgpos = carry
        warea = jnp.maximum(wy2 - wy1, 0.0) * jnp.maximum(wx2 - wx1, 0.0)

        def chunk_body(j, mc):
            maxvec, posvec = mc
            off = j * 16
            sc = score_v[pl.ds(off, 16)]
            cy1 = y1_v[pl.ds(off, 16)]
            cx1 = x1_v[pl.ds(off, 16)]
            cy2 = y2_v[pl.ds(off, 16)]
            cx2 = x2_v[pl.ds(off, 16)]
            ar = area_v[pl.ds(off, 16)]
            yy1 = jnp.maximum(wy1, cy1)
            xx1 = jnp.maximum(wx1, cx1)
            yy2 = jnp.minimum(wy2, cy2)
            xx2 = jnp.minimum(wx2, cx2)
            inter = jnp.maximum(yy2 - yy1, 0.0) * jnp.maximum(xx2 - xx1, 0.0)
            union = warea + ar - inter
            iou = jnp.where(union > 0.0, inter / union, 0.0)
            pos = lane + (base + off)
            sup = (iou > _NMS_THRESHOLD) | (pos == wgpos)
            nsc = jnp.where(sup, _NEG, sc)
            score_v[pl.ds(off, 16)] = nsc
            upd = nsc > maxvec
            return (jnp.where(upd, nsc, maxvec), jnp.where(upd, pos, posvec))

        maxvec0 = jnp.full((16,), _NEG, jnp.float32)
        posvec0 = jnp.full((16,), base, jnp.int32)
        maxvec, posvec = lax.fori_loop(0, _CHUNKS, chunk_body,
                                       (maxvec0, posvec0), unroll=4)
        lmv = _bcast_last(plsc.cummax(maxvec))
        posm = jnp.where(maxvec == lmv, posvec, _BIGPOS)
        lpv = -_bcast_last(plsc.cummax(-posm))
        liv = lpv - base
        stage = jnp.where(lane == 0, _f2i(lmv), lpv)
        stage = jnp.where(lane == 2, _f2i(_rep(y1_v, liv)), stage)
        stage = jnp.where(lane == 3, _f2i(_rep(x1_v, liv)), stage)
        stage = jnp.where(lane == 4, _f2i(_rep(y2_v, liv)), stage)
        stage = jnp.where(lane == 5, _f2i(_rep(x2_v, liv)), stage)
        stage = jnp.where(lane == 1, lpv, stage)
        stage_v[...] = stage

        pltpu.sync_copy(stage_v, shared.at[0, pl.ds(s * 16, 16)])
        plsc.subcore_barrier()
        g0 = (s // _PARTS) * _PARTS
        pltpu.sync_copy(shared.at[0, pl.ds(g0 * 16, _PARTS * 16)], cand_v)
        plsc.subcore_barrier()

        idx0 = jnp.full((16,), 0, jnp.int32)
        idx1 = jnp.full((16,), 1, jnp.int32)

        def _fld(vec, idx):
            return vec.at[idx].get(mode="promise_in_bounds")

        best = cand_v[pl.ds(0, 16)]
        bm = _i2f(_fld(best, idx0))
        bp = _fld(best, idx1)
        for j in range(1, _PARTS):
            cj = cand_v[pl.ds(j * 16, 16)]
            cm = _i2f(_fld(cj, idx0))
            cp = _fld(cj, idx1)
            better = (cm > bm) | ((cm == bm) & (cp < bp))
            bm = jnp.where(better, cm, bm)
            bp = jnp.where(better, cp, bp)
            best = jnp.where(better, cj, best)
        def _field(i):
            idx = jnp.full((16,), i, jnp.int32)
            return best.at[idx].get(mode="promise_in_bounds")

        valid = _i2f(_field(0)) > _NEG
        wgposn = _field(1)
        wy1n = _i2f(_field(2))
        wx1n = _i2f(_field(3))
        wy2n = _i2f(_field(4))
        wx2n = _i2f(_field(5))

        @pl.when(part == 0)
        def _():
            cidx = jnp.minimum(lane + 2, 15)
            coords = _i2f(best.at[cidx].get(mode="promise_in_bounds"))
            vals = jnp.where(valid, coords, -1.0)
            plsc.store_scatter(out_v, [4 * k + lane], vals, mask=lane < 4)

        return (wy1n, wx1n, wy2n, wx2n, wgposn)

    z = jnp.zeros((16,), jnp.float32)
    init = (z, z, z, z, jnp.full((16,), -1, jnp.int32))
    lax.fori_loop(0, _BBOX_NUM, iter_body, init)

    @pl.when(part == 0)
    def _():
        pltpu.sync_copy(out_v, out_hbm.at[row])


@jax.jit
def _nms_sc(scores, y1, x1, y2, x2):
    mesh = plsc.VectorSubcoreMesh(core_axis_name="c", subcore_axis_name="s")
    return pl.kernel(
        _sc_body,
        out_type=jax.ShapeDtypeStruct((_B, _BBOX_NUM * 4), jnp.float32),
        mesh=mesh,
        compiler_params=pltpu.CompilerParams(needs_layout_passes=False),
        scratch_types=[
            pltpu.VMEM((_SHARD,), jnp.float32),
            pltpu.VMEM((_SHARD,), jnp.float32),
            pltpu.VMEM((_SHARD,), jnp.float32),
            pltpu.VMEM((_SHARD,), jnp.float32),
            pltpu.VMEM((_SHARD,), jnp.float32),
            pltpu.VMEM((_SHARD,), jnp.float32),
            pltpu.VMEM((16,), jnp.int32),
            pltpu.VMEM((_PARTS * 16,), jnp.int32),
            pltpu.VMEM((_BBOX_NUM * 4,), jnp.float32),
            pltpu.VMEM_SHARED((2, 256), jnp.int32),
        ],
    )(scores, y1, x1, y2, x2)


def kernel(classifications, bboxes):
    scores = classifications[:, :, 1]
    scores = jnp.pad(scores, ((0, 0), (0, _N_PAD - _N)), constant_values=_NEG)
    coords = jnp.pad(bboxes, ((0, 0), (0, _N_PAD - _N), (0, 0)))
    y1 = coords[:, :, 0]
    x1 = coords[:, :, 1]
    y2 = coords[:, :, 2]
    x2 = coords[:, :, 3]
    out = _nms_sc(scores, y1, x1, y2, x2)
    return out.reshape(_B, _BBOX_NUM, 4)
